# Initial kernel scaffold; baseline (speedup 1.0000x reference)
import jax
import jax.numpy as jnp
from jax.experimental import pallas as pl


def kernel(x, W1a, b1a, W1b, b1b, W2a, b2a, W2b, b2b):
    raise NotImplementedError("write your pallas kernel here")


# speedup vs baseline: 1234.0105x; 1234.0105x over previous
import jax
import jax.numpy as jnp
from jax.experimental import pallas as pl

N = 512
D = 256
H = 256


def _collapsed_gin_kernel(x_ref, w1a_ref, b1a_ref, w1b_ref, b1b_ref,
                          w2a_ref, b2a_ref, w2b_ref, b2b_ref, out_ref):
    s = jnp.sum(x_ref[...], axis=0, keepdims=True)
    h = jnp.maximum(
        jnp.dot(s, w1a_ref[...], preferred_element_type=jnp.float32)
        + b1a_ref[...], 0.0)
    v1 = jnp.maximum(
        jnp.dot(h, w1b_ref[...], preferred_element_type=jnp.float32)
        + b1b_ref[...], 0.0)
    s2 = v1 * jnp.float32(N)
    h2 = jnp.maximum(
        jnp.dot(s2, w2a_ref[...], preferred_element_type=jnp.float32)
        + b2a_ref[...], 0.0)
    v2 = jnp.maximum(
        jnp.dot(h2, w2b_ref[...], preferred_element_type=jnp.float32)
        + b2b_ref[...], 0.0)
    out_ref[...] = v2 * jnp.float32(N)


def kernel(x, W1a, b1a, W1b, b1b, W2a, b2a, W2b, b2b):
    out = pl.pallas_call(
        _collapsed_gin_kernel,
        out_shape=jax.ShapeDtypeStruct((1, D), jnp.float32),
    )(x, W1a, b1a.reshape(1, H), W1b, b1b.reshape(1, D),
      W2a, b2a.reshape(1, H), W2b, b2b.reshape(1, D))
    return out.reshape(D)


# speedup vs baseline: 1247.2186x; 1.0107x over previous
import jax
import jax.numpy as jnp
from jax.experimental import pallas as pl


def _collapsed_gin_kernel(x_ref, w1a_ref, b1a_ref, w1b_ref, b1b_ref,
                          w2a_ref, b2a_ref, w2b_ref, b2b_ref, out_ref):
    n = jnp.float32(x_ref.shape[0])
    s = jnp.sum(x_ref[...], axis=0, keepdims=True)
    h = jnp.maximum(
        jnp.dot(s, w1a_ref[...], preferred_element_type=jnp.float32)
        + b1a_ref[...], 0.0)
    v1 = jnp.maximum(
        jnp.dot(h, w1b_ref[...], preferred_element_type=jnp.float32)
        + b1b_ref[...], 0.0)
    s2 = v1 * n
    h2 = jnp.maximum(
        jnp.dot(s2, w2a_ref[...], preferred_element_type=jnp.float32)
        + b2a_ref[...], 0.0)
    v2 = jnp.maximum(
        jnp.dot(h2, w2b_ref[...], preferred_element_type=jnp.float32)
        + b2b_ref[...], 0.0)
    out_ref[...] = v2 * n


def kernel(x, W1a, b1a, W1b, b1b, W2a, b2a, W2b, b2b):
    d_out = W2b.shape[1]
    out = pl.pallas_call(
        _collapsed_gin_kernel,
        out_shape=jax.ShapeDtypeStruct((1, d_out), jnp.float32),
    )(x, W1a, b1a.reshape(1, -1), W1b, b1b.reshape(1, -1),
      W2a, b2a.reshape(1, -1), W2b, b2b.reshape(1, -1))
    return out.reshape(d_out)
